# baseline (speedup 1.0000x reference)
import functools

import jax
import jax.numpy as jnp
from jax import lax
from jax.experimental import pallas as pl
from jax.experimental.pallas import tpu as pltpu
from jax.experimental.pallas import tpu_sc as plsc

BETA = 0.25


def _vq_body(z1_ref, z2_ref, e_ref, idx_ref, dp_ref):
    Ew = e_ref[...]
    e_sq = jnp.sum(Ew * Ew, axis=1)
    d = None
    for z_ref in (z1_ref, z2_ref):
        zb = z_ref[0]
        m = jax.lax.dot_general(
            Ew, zb, (((1,), (0,)), ((), ())),
            preferred_element_type=jnp.float32)
        zsq = jnp.sum(zb.T * zb.T, axis=1)
        dist = (e_sq[:, None] + zsq[None, :]) - 2.0 * m
        d = dist if d is None else d + dist
    K, P = d.shape
    dmin = jnp.min(d, axis=0)
    iota = jax.lax.broadcasted_iota(jnp.int32, (K, P), 0)
    idx = jnp.min(jnp.where(d == dmin[None, :], iota, K), axis=0)
    idx_ref[0, 0] = idx
    dp_ref[0, 0] = jnp.full((128,), jnp.sum(dmin) * 0.5, jnp.float32)


_C_HALF = 32
_LANES = 16


def _sc_gather_body(et_hbm, idx_hbm, out_hbm, et_v, idx_v, rows_v):
    wid = lax.axis_index("s") * 2 + lax.axis_index("c")
    CK = et_v.shape[0]
    P = idx_v.shape[0]
    K = CK // (2 * _C_HALF)
    pltpu.sync_copy(et_hbm, et_v)
    pltpu.sync_copy(idx_hbm.at[pl.ds(wid * P, P)], idx_v)
    n_g = P // _LANES
    for h in range(2):

        def g_body(g, _, h=h):
            idx_g = idx_v[pl.ds(g * _LANES, _LANES)]
            for c in range(_C_HALF):
                flat = idx_g + jnp.int32((h * _C_HALF + c) * K)
                vals = plsc.load_gather(et_v, [flat])
                rows_v[pl.ds(c * P + g * _LANES, _LANES)] = vals
            return 0

        lax.fori_loop(0, n_g, g_body, 0)
        pltpu.sync_copy(rows_v,
                        out_hbm.at[wid, pl.ds(h * _C_HALF * P, _C_HALF * P)])


def kernel(z_e_1, z_e_2, E):
    B, C, H, W = z_e_1.shape
    P = H * W
    K = E.shape[0]
    z1 = z_e_1.reshape(B, C, P)
    z2 = z_e_2.reshape(B, C, P)

    idx, dparts = pl.pallas_call(
        _vq_body,
        grid=(B,),
        in_specs=[
            pl.BlockSpec((1, C, P), lambda b: (b, 0, 0)),
            pl.BlockSpec((1, C, P), lambda b: (b, 0, 0)),
            pl.BlockSpec((K, C), lambda b: (0, 0)),
        ],
        out_specs=[
            pl.BlockSpec((1, 1, P), lambda b: (b, 0, 0)),
            pl.BlockSpec((1, 1, 128), lambda b: (b, 0, 0)),
        ],
        out_shape=[
            jax.ShapeDtypeStruct((B, 1, P), jnp.int32),
            jax.ShapeDtypeStruct((B, 1, 128), jnp.float32),
        ],
    )(z1, z2, E)

    indices = idx.reshape(B * P)

    sc_gather = functools.partial(
        pl.kernel,
        mesh=plsc.VectorSubcoreMesh(core_axis_name="c", subcore_axis_name="s"),
        compiler_params=pltpu.CompilerParams(needs_layout_passes=False),
        out_type=jax.ShapeDtypeStruct((B, C * P), jnp.float32),
        scratch_types=[
            pltpu.VMEM((C * K,), jnp.float32),
            pltpu.VMEM((P,), jnp.int32),
            pltpu.VMEM((_C_HALF * P,), jnp.float32),
        ],
    )(_sc_gather_body)
    z_q = sc_gather(E.T.reshape(C * K), indices).reshape(B, C, H, W)
    n_el = float(B * C * H * W)
    codebook_loss = jnp.sum(dparts[:, 0, 0]) / n_el
    commitment_loss = codebook_loss
    vq_loss = codebook_loss + BETA * commitment_loss
    return (z_q, codebook_loss, commitment_loss, vq_loss, indices)


# speedup vs baseline: 1.0399x; 1.0399x over previous
import functools

import jax
import jax.numpy as jnp
from jax import lax
from jax.experimental import pallas as pl
from jax.experimental.pallas import tpu as pltpu
from jax.experimental.pallas import tpu_sc as plsc

BETA = 0.25


def _vq_body(z1_ref, z2_ref, e_ref, idx_ref, dp_ref):
    Ew = e_ref[...]
    e_sq = jnp.sum(Ew * Ew, axis=1)
    d = None
    for z_ref in (z1_ref, z2_ref):
        zb = z_ref[0]
        m = jax.lax.dot_general(
            Ew, zb, (((1,), (0,)), ((), ())),
            preferred_element_type=jnp.float32)
        zsq = jnp.sum(zb.T * zb.T, axis=1)
        dist = (e_sq[:, None] + zsq[None, :]) - 2.0 * m
        d = dist if d is None else d + dist
    K, P = d.shape
    dmin = jnp.min(d, axis=0)
    iota = jax.lax.broadcasted_iota(jnp.int32, (K, P), 0)
    idx = jnp.min(jnp.where(d == dmin[None, :], iota, K), axis=0)
    idx_ref[0, 0] = idx
    dp_ref[0, 0] = jnp.full((128,), jnp.sum(dmin) * 0.5, jnp.float32)


_C_HALF = 32
_LANES = 16


def _sc_gather_body(et_hbm, idx_hbm, out_hbm, et_v, idx_v, rows_v):
    wid = lax.axis_index("s") * 2 + lax.axis_index("c")
    CK = et_v.shape[0]
    P = idx_v.shape[0]
    K = CK // (2 * _C_HALF)
    pltpu.sync_copy(et_hbm, et_v)
    pltpu.sync_copy(idx_hbm.at[pl.ds(wid * P, P)], idx_v)
    n_g = P // _LANES
    for h in range(2):

        @plsc.parallel_loop(0, n_g, unroll=4)
        def _g_body(g, h=h):
            idx_g = idx_v[pl.ds(g * _LANES, _LANES)]
            for c in range(_C_HALF):
                flat = idx_g + jnp.int32((h * _C_HALF + c) * K)
                vals = plsc.load_gather(et_v, [flat])
                rows_v[pl.ds(c * P + g * _LANES, _LANES)] = vals

        pltpu.sync_copy(rows_v,
                        out_hbm.at[wid, pl.ds(h * _C_HALF * P, _C_HALF * P)])


def kernel(z_e_1, z_e_2, E):
    B, C, H, W = z_e_1.shape
    P = H * W
    K = E.shape[0]
    z1 = z_e_1.reshape(B, C, P)
    z2 = z_e_2.reshape(B, C, P)

    idx, dparts = pl.pallas_call(
        _vq_body,
        grid=(B,),
        in_specs=[
            pl.BlockSpec((1, C, P), lambda b: (b, 0, 0)),
            pl.BlockSpec((1, C, P), lambda b: (b, 0, 0)),
            pl.BlockSpec((K, C), lambda b: (0, 0)),
        ],
        out_specs=[
            pl.BlockSpec((1, 1, P), lambda b: (b, 0, 0)),
            pl.BlockSpec((1, 1, 128), lambda b: (b, 0, 0)),
        ],
        out_shape=[
            jax.ShapeDtypeStruct((B, 1, P), jnp.int32),
            jax.ShapeDtypeStruct((B, 1, 128), jnp.float32),
        ],
    )(z1, z2, E)

    indices = idx.reshape(B * P)

    sc_gather = functools.partial(
        pl.kernel,
        mesh=plsc.VectorSubcoreMesh(core_axis_name="c", subcore_axis_name="s"),
        compiler_params=pltpu.CompilerParams(needs_layout_passes=False),
        out_type=jax.ShapeDtypeStruct((B, C * P), jnp.float32),
        scratch_types=[
            pltpu.VMEM((C * K,), jnp.float32),
            pltpu.VMEM((P,), jnp.int32),
            pltpu.VMEM((_C_HALF * P,), jnp.float32),
        ],
    )(_sc_gather_body)
    z_q = sc_gather(E.T.reshape(C * K), indices).reshape(B, C, H, W)
    n_el = float(B * C * H * W)
    codebook_loss = jnp.sum(dparts[:, 0, 0]) / n_el
    commitment_loss = codebook_loss
    vq_loss = codebook_loss + BETA * commitment_loss
    return (z_q, codebook_loss, commitment_loss, vq_loss, indices)


# speedup vs baseline: 1.4169x; 1.3626x over previous
import jax
import jax.numpy as jnp
from jax.experimental import pallas as pl

BETA = 0.25


def _vq_body(z1_ref, z2_ref, e_ref, zq_ref, idx_ref, dp_ref):
    Ew = e_ref[...]
    e_sq = jnp.sum(Ew * Ew, axis=1)
    d = None
    for z_ref in (z1_ref, z2_ref):
        zb = z_ref[0]
        m = jax.lax.dot_general(
            Ew, zb, (((1,), (0,)), ((), ())),
            preferred_element_type=jnp.float32)
        zsq = jnp.sum(zb.T * zb.T, axis=1)
        dist = (e_sq[:, None] + zsq[None, :]) - 2.0 * m
        d = dist if d is None else d + dist
    K, P = d.shape
    dmin = jnp.min(d, axis=0)
    iota = jax.lax.broadcasted_iota(jnp.int32, (K, P), 0)
    idx = jnp.min(jnp.where(d == dmin[None, :], iota, K), axis=0)
    oh = (iota == idx[None, :]).astype(jnp.bfloat16)
    zq = jax.lax.dot_general(
        Ew.astype(jnp.bfloat16), oh, (((0,), (0,)), ((), ())),
        preferred_element_type=jnp.float32)
    zq_ref[0] = zq
    idx_ref[0, 0] = idx
    dp_ref[0, 0] = jnp.full((128,), jnp.sum(dmin) * 0.5, jnp.float32)


def kernel(z_e_1, z_e_2, E):
    B, C, H, W = z_e_1.shape
    P = H * W
    K = E.shape[0]
    z1 = z_e_1.reshape(B, C, P)
    z2 = z_e_2.reshape(B, C, P)

    zq, idx, dparts = pl.pallas_call(
        _vq_body,
        grid=(B,),
        in_specs=[
            pl.BlockSpec((1, C, P), lambda b: (b, 0, 0)),
            pl.BlockSpec((1, C, P), lambda b: (b, 0, 0)),
            pl.BlockSpec((K, C), lambda b: (0, 0)),
        ],
        out_specs=[
            pl.BlockSpec((1, C, P), lambda b: (b, 0, 0)),
            pl.BlockSpec((1, 1, P), lambda b: (b, 0, 0)),
            pl.BlockSpec((1, 1, 128), lambda b: (b, 0, 0)),
        ],
        out_shape=[
            jax.ShapeDtypeStruct((B, C, P), jnp.float32),
            jax.ShapeDtypeStruct((B, 1, P), jnp.int32),
            jax.ShapeDtypeStruct((B, 1, 128), jnp.float32),
        ],
    )(z1, z2, E)

    z_q = zq.reshape(B, C, H, W)
    indices = idx.reshape(B * P)
    n_el = float(B * C * H * W)
    codebook_loss = jnp.sum(dparts[:, 0, 0]) / n_el
    commitment_loss = codebook_loss
    vq_loss = codebook_loss + BETA * commitment_loss
    return (z_q, codebook_loss, commitment_loss, vq_loss, indices)
